# baseline (speedup 1.0000x reference)
import jax
import jax.numpy as jnp
from jax.experimental import pallas as pl
from jax.experimental.pallas import tpu as pltpu

_TILE = 200
_NBUF = 4


_NSPLIT = 5
_SUB = _TILE // _NSPLIT


def _strip_copies(adj_hbm, buf, sem, step, slot):
    return [
        pltpu.make_async_copy(
            adj_hbm.at[pl.ds(step * _TILE + k * _SUB, _SUB), :],
            buf.at[slot, pl.ds(k * _SUB, _SUB), :],
            sem.at[slot, k],
        )
        for k in range(_NSPLIT)
    ]


def _gcn_kernel(seq_ref, w_ref, bias_ref, a_ref, adj_hbm, out_ref,
                fts_ref, buf, sem):
    i = pl.program_id(0)
    nsteps = pl.num_programs(0)

    @pl.when(i == 0)
    def _init():
        for j in range(_NBUF):
            for c in _strip_copies(adj_hbm, buf, sem, j, j):
                c.start()
        fts_ref[...] = jax.lax.dot_general(
            seq_ref[...], w_ref[...],
            (((1,), (1,)), ((), ())),
            preferred_element_type=jnp.float32,
        )

    slot = jax.lax.rem(i, _NBUF)
    for c in _strip_copies(adj_hbm, buf, sem, i, slot):
        c.wait()
    acc = jnp.dot(buf[slot], fts_ref[...], preferred_element_type=jnp.float32)
    acc = acc + bias_ref[...]
    out_ref[...] = jnp.where(acc > 0, acc, a_ref[0, 0] * acc)

    nxt = i + _NBUF

    @pl.when(nxt < nsteps)
    def _prefetch():
        for c in _strip_copies(adj_hbm, buf, sem, nxt, slot):
            c.start()


def kernel(seq, adj, W, bias, prelu_a):
    b, n, d_in = seq.shape
    d_out = W.shape[0]
    seq2 = seq.reshape(n, d_in)
    adj2 = adj.reshape(n, n)
    bias2 = bias.reshape(1, d_out)
    a2 = prelu_a.reshape(1, 1)

    grid = (n // _TILE,)

    out = pl.pallas_call(
        _gcn_kernel,
        grid=grid,
        in_specs=[
            pl.BlockSpec((n, d_in), lambda i: (0, 0)),
            pl.BlockSpec((d_out, d_in), lambda i: (0, 0)),
            pl.BlockSpec((1, d_out), lambda i: (0, 0)),
            pl.BlockSpec((1, 1), lambda i: (0, 0)),
            pl.BlockSpec(memory_space=pl.ANY),
        ],
        out_specs=pl.BlockSpec((_TILE, d_out), lambda i: (i, 0)),
        out_shape=jax.ShapeDtypeStruct((n, d_out), jnp.float32),
        scratch_shapes=[
            pltpu.VMEM((n, d_in), jnp.float32),
            pltpu.VMEM((_NBUF, _TILE, n), jnp.float32),
            pltpu.SemaphoreType.DMA((_NBUF, _NSPLIT)),
        ],
    )(seq2, W, bias2, a2, adj2)
    return out.reshape(b, n, d_out)


# speedup vs baseline: 1.0113x; 1.0113x over previous
import jax
import jax.numpy as jnp
from jax.experimental import pallas as pl
from jax.experimental.pallas import tpu as pltpu


def _gcn_kernel(seq_ref, w_ref, bias_ref, a_ref, adj_ref, out_ref, fts_ref):
    i = pl.program_id(0)

    @pl.when(i == 0)
    def _compute_fts():
        fts_ref[...] = jax.lax.dot_general(
            seq_ref[...], w_ref[...],
            (((1,), (1,)), ((), ())),
            preferred_element_type=jnp.float32,
        )

    acc = jnp.dot(adj_ref[...], fts_ref[...], preferred_element_type=jnp.float32)
    acc = acc + bias_ref[...]
    out_ref[...] = jnp.where(acc > 0, acc, a_ref[0, 0] * acc)


def kernel(seq, adj, W, bias, prelu_a):
    b, n, d_in = seq.shape
    d_out = W.shape[0]
    seq2 = seq.reshape(n, d_in)
    adj2 = adj.reshape(n, n)
    bias2 = bias.reshape(1, d_out)
    a2 = prelu_a.reshape(1, 1)

    tile = 200
    grid = (n // tile,)

    out = pl.pallas_call(
        _gcn_kernel,
        grid=grid,
        in_specs=[
            pl.BlockSpec((n, d_in), lambda i: (0, 0)),
            pl.BlockSpec((d_out, d_in), lambda i: (0, 0)),
            pl.BlockSpec((1, d_out), lambda i: (0, 0)),
            pl.BlockSpec((1, 1), lambda i: (0, 0)),
            pl.BlockSpec((tile, n), lambda i: (i, 0)),
        ],
        out_specs=pl.BlockSpec((tile, d_out), lambda i: (i, 0)),
        out_shape=jax.ShapeDtypeStruct((n, d_out), jnp.float32),
        scratch_shapes=[pltpu.VMEM((n, d_in), jnp.float32)],
    )(seq2, W, bias2, a2, adj2)
    return out.reshape(b, n, d_out)
